# baseline (speedup 1.0000x reference)
import functools

import jax
import jax.numpy as jnp
from jax import lax
from jax.experimental import pallas as pl
from jax.experimental.pallas import tpu as pltpu
from jax.experimental.pallas import tpu_sc as plsc

_LST = 64
_NB = 8


def _bisect_tau(z, axis):
    d = float(z.shape[axis])
    hi = jnp.max(z, axis=axis, keepdims=True)
    lo = hi - 1.0
    for _ in range(_NB):
        mid = 0.5 * (lo + hi)
        g = jnp.sum(jnp.maximum(z, mid), axis=axis, keepdims=True)
        ge = g >= 1.0 + d * mid
        lo = jnp.where(ge, mid, lo)
        hi = jnp.where(ge, hi, mid)
    sup = (z > lo).astype(jnp.float32)
    c = jnp.sum(sup, axis=axis, keepdims=True)
    s = jnp.sum(z * sup, axis=axis, keepdims=True)
    return jnp.clip((s - 1.0) / c, lo, hi)


def _body(x_ref, o_ref, *, t_block):
    z = x_ref[0]
    n_insts = z.shape[0]
    tau_i = _bisect_tau(z, axis=0)
    pi = jnp.maximum(z - tau_i, 0.0)
    nf = t_block // _LST
    zt = z.T.reshape(nf, _LST, n_insts)
    tau_t = _bisect_tau(zt, axis=1)
    tau_b = jnp.repeat(tau_t.reshape(nf, n_insts).T, _LST, axis=1)
    pt = jnp.maximum(z - tau_b, 0.0)
    o_ref[0] = pi * pt


def _tc_call(x):
    batch, n_insts, time = x.shape
    t_block = 4096
    if time % t_block:
        t_block = _LST
    grid = (batch, time // t_block)
    spec = pl.BlockSpec((1, n_insts, t_block), lambda b, t: (b, 0, t))
    return pl.pallas_call(
        functools.partial(_body, t_block=t_block),
        grid=grid,
        in_specs=[spec],
        out_specs=spec,
        out_shape=jax.ShapeDtypeStruct(x.shape, x.dtype),
    )(x)


def kernel(midis_out):
    return _tc_call(midis_out)


# speedup vs baseline: 1.4494x; 1.4494x over previous
import functools

import jax
import jax.numpy as jnp
from jax import lax
from jax.experimental import pallas as pl
from jax.experimental.pallas import tpu as pltpu
from jax.experimental.pallas import tpu_sc as plsc

_LST = 64
_NB = 7


def _bisect_tau(z, axis):
    d = float(z.shape[axis])
    hi = jnp.max(z, axis=axis, keepdims=True)
    lo = hi - 1.0
    for _ in range(_NB):
        mid = 0.5 * (lo + hi)
        g = jnp.sum(jnp.maximum(z, mid), axis=axis, keepdims=True)
        ge = g >= 1.0 + d * mid
        lo = jnp.where(ge, mid, lo)
        hi = jnp.where(ge, hi, mid)
    sup = (z > lo).astype(jnp.float32)
    c = jnp.sum(sup, axis=axis, keepdims=True)
    s = jnp.sum(z * sup, axis=axis, keepdims=True)
    return jnp.clip((s - 1.0) / c, lo, hi)


def _body(x_ref, o_ref, *, t_block):
    z = x_ref[0]
    n_insts = z.shape[0]
    tau_i = _bisect_tau(z, axis=0)
    pi = jnp.maximum(z - tau_i, 0.0)
    nf = t_block // _LST
    zt = z.T.reshape(nf, _LST, n_insts)
    tau_t = _bisect_tau(zt, axis=1)
    pt = jnp.maximum(zt - tau_t, 0.0).reshape(t_block, n_insts).T
    o_ref[0] = pi * pt


def _tc_call(x):
    batch, n_insts, time = x.shape
    t_block = 4096
    if time % t_block:
        t_block = _LST
    grid = (batch, time // t_block)
    spec = pl.BlockSpec((1, n_insts, t_block), lambda b, t: (b, 0, t))
    return pl.pallas_call(
        functools.partial(_body, t_block=t_block),
        grid=grid,
        in_specs=[spec],
        out_specs=spec,
        out_shape=jax.ShapeDtypeStruct(x.shape, x.dtype),
    )(x)


def kernel(midis_out):
    return _tc_call(midis_out)


# speedup vs baseline: 1.6050x; 1.1074x over previous
import functools

import jax
import jax.numpy as jnp
from jax import lax
from jax.experimental import pallas as pl
from jax.experimental.pallas import tpu as pltpu
from jax.experimental.pallas import tpu_sc as plsc

_LST = 64
_NB = 6


def _bisect_tau(z, axis):
    d = float(z.shape[axis])
    hi = jnp.max(z, axis=axis, keepdims=True)
    lo = hi - 1.0
    for _ in range(_NB):
        mid = 0.5 * (lo + hi)
        g = jnp.sum(jnp.maximum(z, mid), axis=axis, keepdims=True)
        ge = g >= 1.0 + d * mid
        lo = jnp.where(ge, mid, lo)
        hi = jnp.where(ge, hi, mid)
    sup = (z > lo).astype(jnp.float32)
    c = jnp.sum(sup, axis=axis, keepdims=True)
    s = jnp.sum(z * sup, axis=axis, keepdims=True)
    return jnp.clip((s - 1.0) / c, lo, hi)


def _body(x_ref, o_ref, *, t_block):
    z = x_ref[0]
    n_insts = z.shape[0]
    tau_i = _bisect_tau(z, axis=0)
    pi = jnp.maximum(z - tau_i, 0.0)
    nf = t_block // _LST
    zt = z.T.reshape(nf, _LST, n_insts)
    tau_t = _bisect_tau(zt, axis=1)
    pt = jnp.maximum(zt - tau_t, 0.0).reshape(t_block, n_insts).T
    o_ref[0] = pi * pt


def _tc_call(x):
    batch, n_insts, time = x.shape
    t_block = 4096
    if time % t_block:
        t_block = _LST
    grid = (batch, time // t_block)
    spec = pl.BlockSpec((1, n_insts, t_block), lambda b, t: (b, 0, t))
    return pl.pallas_call(
        functools.partial(_body, t_block=t_block),
        grid=grid,
        in_specs=[spec],
        out_specs=spec,
        out_shape=jax.ShapeDtypeStruct(x.shape, x.dtype),
    )(x)


def kernel(midis_out):
    return _tc_call(midis_out)


# speedup vs baseline: 1.6054x; 1.0002x over previous
import functools

import jax
import jax.numpy as jnp
from jax.experimental import pallas as pl

_LST = 64
_NB = 6


def _bisect_tau(z, axis):
    d = float(z.shape[axis])
    hi = jnp.max(z, axis=axis, keepdims=True)
    lo = hi - 1.0
    for _ in range(_NB):
        mid = 0.5 * (lo + hi)
        g = jnp.sum(jnp.maximum(z, mid), axis=axis, keepdims=True)
        ge = g >= 1.0 + d * mid
        lo = jnp.where(ge, mid, lo)
        hi = jnp.where(ge, hi, mid)
    sup = (z > lo).astype(jnp.float32)
    c = jnp.sum(sup, axis=axis, keepdims=True)
    s = jnp.sum(z * sup, axis=axis, keepdims=True)
    return jnp.clip((s - 1.0) / c, lo, hi)


def _body(x_ref, o_ref, *, t_block):
    z = x_ref[0]
    n_insts = z.shape[0]
    tau_i = _bisect_tau(z, axis=0)
    pi = jnp.maximum(z - tau_i, 0.0)
    nf = t_block // _LST
    zt = z.T.reshape(nf, _LST, n_insts)
    tau_t = _bisect_tau(zt, axis=1)
    pt = jnp.maximum(zt - tau_t, 0.0).reshape(t_block, n_insts).T
    o_ref[0] = pi * pt


def _tc_call(x):
    batch, n_insts, time = x.shape
    t_block = 4096
    if time % t_block:
        t_block = _LST
    grid = (batch, time // t_block)
    spec = pl.BlockSpec((1, n_insts, t_block), lambda b, t: (b, 0, t))
    return pl.pallas_call(
        functools.partial(_body, t_block=t_block),
        grid=grid,
        in_specs=[spec],
        out_specs=spec,
        out_shape=jax.ShapeDtypeStruct(x.shape, x.dtype),
    )(x)


def kernel(midis_out):
    return _tc_call(midis_out)
